# baseline (speedup 1.0000x reference)
import functools

import jax
import jax.numpy as jnp
from jax import lax
from jax.experimental import pallas as pl
from jax.experimental.pallas import tpu as pltpu
from jax.experimental.pallas import tpu_sc as plsc

_N = 10000
_D = 128
_E_RAW = 320000
_E_TOT = _E_RAW + _N
_NW = 32
_BATCH = 128
_NB = 82
_E_PAD = _NW * _NB * _BATCH
_N_PAD = 10240
_RPT = _N_PAD // 16
_N_ROW_PAD = 10112
_RPT2 = _N_ROW_PAD // 16

_mesh = plsc.VectorSubcoreMesh(core_axis_name="c", subcore_axis_name="s")


@functools.partial(
    pl.kernel,
    mesh=_mesh,
    compiler_params=pltpu.CompilerParams(needs_layout_passes=False),
    out_type=jax.ShapeDtypeStruct((2, _N_ROW_PAD, _D), jnp.float32),
    scratch_types=[
        pltpu.VMEM((_NB, _BATCH), jnp.int32),
        pltpu.VMEM((_BATCH,), jnp.int32),
        pltpu.VMEM((_BATCH,), jnp.int32),
        pltpu.VMEM((_BATCH,), jnp.float32),
        pltpu.VMEM((_BATCH,), jnp.float32),
        pltpu.VMEM((_BATCH, _D), jnp.float32),
        pltpu.VMEM((_BATCH, _D), jnp.float32),
        pltpu.VMEM_SHARED((_N_ROW_PAD, _D), jnp.float32),
        pltpu.SemaphoreType.DMA,
        pltpu.SemaphoreType.DMA,
        pltpu.SemaphoreType.DMA,
        pltpu.SemaphoreType.DMA,
    ],
)
def _sc_segsum_rows(h_hbm, src_hbm, dst_hbm, w_hbm, zero_hbm, part_hbm,
                    src_v, dst0_v, dst1_v, w0_v, w1_v, rows0_v, rows1_v,
                    acc_sh, sem0, sem1, semdw0, semdw1):
    c = lax.axis_index("c")
    s = lax.axis_index("s")
    wid = c * 16 + s
    pltpu.sync_copy(src_hbm.at[wid], src_v)
    pltpu.sync_copy(zero_hbm.at[pl.ds(s * _RPT2, _RPT2)],
                    acc_sh.at[pl.ds(s * _RPT2, _RPT2)])
    plsc.subcore_barrier()

    def scale(rows_v, w_v):
        def edge_body(j4, carry2):
            j = j4 * 4
            for u in range(4):
                wsp = plsc.load_gather(w_v, [jnp.full((16,), j + u, jnp.int32)])
                for t in range(_D // 16):
                    rows_v[j + u, pl.ds(t * 16, 16)] = (
                        rows_v[j + u, pl.ds(t * 16, 16)] * wsp)
            return carry2
        lax.fori_loop(0, _BATCH // 4, edge_body, 0)

    def issue_dw(b, dst_v, w_v, sem):
        pltpu.async_copy(dst_hbm.at[wid, b], dst_v, sem)
        pltpu.async_copy(w_hbm.at[wid, b], w_v, sem)

    def wait_dw(b, dst_v, w_v, sem):
        pltpu.make_async_copy(dst_hbm.at[wid, b], dst_v, sem).wait()
        pltpu.make_async_copy(w_hbm.at[wid, b], w_v, sem).wait()

    pltpu.async_copy(h_hbm.at[src_v.at[0]], rows0_v, sem0)
    issue_dw(0, dst0_v, w0_v, semdw0)
    issue_dw(1, dst1_v, w1_v, semdw1)

    def batch_body(g, carry):
        b0 = 2 * g
        b1 = 2 * g + 1
        pltpu.make_async_copy(h_hbm.at[src_v.at[b0]], rows0_v, sem0).wait()
        pltpu.async_copy(h_hbm.at[src_v.at[b1]], rows1_v, sem1)
        wait_dw(b0, dst0_v, w0_v, semdw0)
        scale(rows0_v, w0_v)
        pltpu.sync_copy(rows0_v, acc_sh.at[dst0_v], add=True)
        pltpu.async_copy(h_hbm.at[src_v.at[(b0 + 2) % _NB]], rows0_v, sem0)
        issue_dw((b0 + 2) % _NB, dst0_v, w0_v, semdw0)
        pltpu.make_async_copy(h_hbm.at[src_v.at[b1]], rows1_v, sem1).wait()
        wait_dw(b1, dst1_v, w1_v, semdw1)
        scale(rows1_v, w1_v)
        pltpu.sync_copy(rows1_v, acc_sh.at[dst1_v], add=True)
        issue_dw((b1 + 2) % _NB, dst1_v, w1_v, semdw1)
        return carry

    lax.fori_loop(0, _NB // 2, batch_body, 0)
    pltpu.make_async_copy(h_hbm.at[src_v.at[0]], rows0_v, sem0).wait()
    wait_dw(0, dst0_v, w0_v, semdw0)
    wait_dw(1, dst1_v, w1_v, semdw1)
    plsc.subcore_barrier()
    pltpu.sync_copy(acc_sh.at[pl.ds(s * _RPT2, _RPT2)],
                    part_hbm.at[c, pl.ds(s * _RPT2, _RPT2)])


@functools.partial(
    pl.kernel,
    mesh=_mesh,
    compiler_params=pltpu.CompilerParams(needs_layout_passes=False),
    out_type=jax.ShapeDtypeStruct((2, _N_PAD), jnp.float32),
    scratch_types=[
        pltpu.VMEM((_NB, _BATCH), jnp.int32),
        pltpu.VMEM((_NB, _BATCH), jnp.float32),
        pltpu.VMEM_SHARED((_N_PAD,), jnp.float32),
    ],
)
def _sc_degree(dst_hbm, w_hbm, zero_hbm, part_hbm, dst_v, w_v, acc_sh):
    c = lax.axis_index("c")
    s = lax.axis_index("s")
    wid = c * 16 + s
    pltpu.sync_copy(dst_hbm.at[wid], dst_v)
    pltpu.sync_copy(w_hbm.at[wid], w_v)
    pltpu.sync_copy(zero_hbm.at[pl.ds(s * _RPT, _RPT)],
                    acc_sh.at[pl.ds(s * _RPT, _RPT)])
    plsc.subcore_barrier()

    def batch_body(b, carry):
        pltpu.sync_copy(w_v.at[b], acc_sh.at[dst_v.at[b]], add=True)
        return carry

    lax.fori_loop(0, _NB, batch_body, 0)
    plsc.subcore_barrier()
    pltpu.sync_copy(acc_sh.at[pl.ds(s * _RPT, _RPT)],
                    part_hbm.at[c, pl.ds(s * _RPT, _RPT)])


@functools.partial(
    pl.kernel,
    mesh=_mesh,
    compiler_params=pltpu.CompilerParams(needs_layout_passes=False),
    out_type=(
        jax.ShapeDtypeStruct((_NW, _NB, _BATCH), jnp.float32),
        jax.ShapeDtypeStruct((2, _N_PAD), jnp.float32),
    ),
    scratch_types=[
        pltpu.VMEM((_N_PAD,), jnp.float32),
        pltpu.VMEM((_N_PAD,), jnp.float32),
        pltpu.VMEM((16,), jnp.float32),
        pltpu.VMEM((_NB, _BATCH), jnp.int32),
        pltpu.VMEM((_NB, _BATCH), jnp.int32),
        pltpu.VMEM((_NB, _BATCH), jnp.float32),
        pltpu.VMEM((_NB, _BATCH), jnp.float32),
        pltpu.VMEM_SHARED((_N_PAD,), jnp.float32),
    ],
)
def _sc_gat_edges(s1_hbm, s2_hbm, shift_hbm, src_hbm, dst_hbm, mask_hbm,
                  zero_hbm, ex_hbm, dpart_hbm,
                  s1_v, s2_v, shift_v, src_v, dst_v, mask_v, ex_v, acc_sh):
    c = lax.axis_index("c")
    s = lax.axis_index("s")
    wid = c * 16 + s
    pltpu.sync_copy(s1_hbm, s1_v)
    pltpu.sync_copy(s2_hbm, s2_v)
    pltpu.sync_copy(shift_hbm, shift_v)
    pltpu.sync_copy(src_hbm.at[wid], src_v)
    pltpu.sync_copy(dst_hbm.at[wid], dst_v)
    pltpu.sync_copy(mask_hbm.at[wid], mask_v)
    pltpu.sync_copy(zero_hbm.at[pl.ds(s * _RPT, _RPT)],
                    acc_sh.at[pl.ds(s * _RPT, _RPT)])
    plsc.subcore_barrier()
    shift = shift_v[...]

    def batch_body(b, carry):
        def grp_body(k, carry2):
            src16 = src_v[b, pl.ds(k * 16, 16)]
            dst16 = dst_v[b, pl.ds(k * 16, 16)]
            m16 = mask_v[b, pl.ds(k * 16, 16)]
            a = plsc.load_gather(s1_v, [src16]) + plsc.load_gather(s2_v, [dst16])
            a = jnp.where(a > 0.0, a, 0.2 * a) - shift
            ex_v[b, pl.ds(k * 16, 16)] = jnp.exp(a) * m16
            return carry2

        lax.fori_loop(0, _BATCH // 16, grp_body, 0)
        pltpu.sync_copy(ex_v.at[b], acc_sh.at[dst_v.at[b]], add=True)
        return carry

    lax.fori_loop(0, _NB, batch_body, 0)
    pltpu.sync_copy(ex_v, ex_hbm.at[wid])
    plsc.subcore_barrier()
    pltpu.sync_copy(acc_sh.at[pl.ds(s * _RPT, _RPT)],
                    dpart_hbm.at[c, pl.ds(s * _RPT, _RPT)])


def _tc1_body(x_ref, w1_ref, degp_ref, hs1_ref, dinv_ref):
    x = x_ref[...]
    x = jnp.where(jnp.isnan(x) | jnp.isinf(x), jnp.zeros_like(x), x)
    deg = (degp_ref[0, :] + degp_ref[1, :])[:_N]
    dinv = jnp.where(deg > 0.0, lax.rsqrt(deg), 0.0)
    dinv_ref[...] = dinv[:, None]
    hs1_ref[...] = dinv[:, None] * jnp.dot(
        x, w1_ref[...], preferred_element_type=jnp.float32)


def _tc2_body(part_ref, dinv_ref, b1_ref, gamma_ref, beta_ref, wg_ref,
              asrc_ref, adst_ref, hg_ref, s1_ref, s2_ref, shift_ref):
    y = dinv_ref[...] * (part_ref[0] + part_ref[1])[:_N] + b1_ref[...]
    mean = jnp.mean(y, axis=0, keepdims=True)
    var = jnp.mean((y - mean) * (y - mean), axis=0, keepdims=True)
    y = (y - mean) * lax.rsqrt(var + 1e-5) * gamma_ref[...] + beta_ref[...]
    y = jnp.maximum(y, 0.0)
    hg = jnp.dot(y, wg_ref[...], preferred_element_type=jnp.float32)
    hg_ref[...] = hg
    s1 = jnp.dot(hg, asrc_ref[...], preferred_element_type=jnp.float32)
    s2 = jnp.dot(hg, adst_ref[...], preferred_element_type=jnp.float32)
    s1_ref[...] = s1
    s2_ref[...] = s2
    shift_ref[...] = jnp.full((1, 1), 0.0, jnp.float32) + jnp.max(s1) + jnp.max(s2)


def _tc3_body(gpart_ref, dpart_ref, bg_ref, w2_ref, dinv_ref, hs2_ref):
    denom = (dpart_ref[0, :] + dpart_ref[1, :])[:_N, None]
    y = (gpart_ref[0] + gpart_ref[1])[:_N] / denom + bg_ref[...]
    y = jnp.maximum(y, 0.0)
    hs2_ref[...] = dinv_ref[...] * jnp.dot(
        y, w2_ref[...], preferred_element_type=jnp.float32)


def _tc4_body(part_ref, dinv_ref, b2_ref, w3_ref, hs3_ref):
    y = dinv_ref[...] * (part_ref[0] + part_ref[1])[:_N] + b2_ref[...]
    hs3_ref[...] = dinv_ref[...] * jnp.dot(
        y, w3_ref[...], preferred_element_type=jnp.float32)


def _tc5_body(part_ref, dinv_ref, b3_ref, out_ref):
    out_ref[...] = dinv_ref[...] * (part_ref[0] + part_ref[1])[:_N] + b3_ref[...]


def _tc(body, out_shape, *args):
    return pl.pallas_call(body, out_shape=out_shape)(*args)


def kernel(x, edge_index, edge_weight, W1, b1, gamma, beta, Wg, att_src,
           att_dst, bg, W2, b2, W3, b3):
    f32 = jnp.float32
    loop = jnp.arange(_N, dtype=edge_index.dtype)
    src = jnp.concatenate([edge_index[0], loop])
    dst = jnp.concatenate([edge_index[1], loop])
    ew = jnp.concatenate([edge_weight, jnp.ones((_N,), f32)])
    pad = _E_PAD - _E_TOT
    src_c = jnp.pad(src, (0, pad)).reshape(_NW, _NB, _BATCH)
    dst_c = jnp.pad(dst, (0, pad)).reshape(_NW, _NB, _BATCH)
    ew_c = jnp.pad(ew, (0, pad)).reshape(_NW, _NB, _BATCH)
    mask_c = jnp.pad(jnp.ones((_E_TOT,), f32), (0, pad)).reshape(_NW, _NB, _BATCH)
    zrow = jnp.zeros((_N_ROW_PAD, _D), f32)
    zvec = jnp.zeros((_N_PAD,), f32)

    deg_parts = _sc_degree(dst_c, ew_c, zvec)

    hs1, dinv = _tc(
        _tc1_body,
        (jax.ShapeDtypeStruct((_N, _D), f32), jax.ShapeDtypeStruct((_N, 1), f32)),
        x, W1, deg_parts)
    p1 = _sc_segsum_rows(hs1, src_c, dst_c, ew_c, zrow)

    hg, s1, s2, shift = _tc(
        _tc2_body,
        (jax.ShapeDtypeStruct((_N, _D), f32),
         jax.ShapeDtypeStruct((_N,), f32),
         jax.ShapeDtypeStruct((_N,), f32),
         jax.ShapeDtypeStruct((1, 1), f32)),
        p1, dinv, b1, gamma, beta, Wg, att_src, att_dst)
    shift16 = jnp.broadcast_to(shift.reshape(()), (16,))

    s1p = jnp.pad(s1, (0, _N_PAD - _N))
    s2p = jnp.pad(s2, (0, _N_PAD - _N))
    ex_c, den_parts = _sc_gat_edges(s1p, s2p, shift16, src_c, dst_c, mask_c, zvec)
    pg = _sc_segsum_rows(hg, src_c, dst_c, ex_c, zrow)

    hs2 = _tc(_tc3_body, jax.ShapeDtypeStruct((_N, _D), f32),
              pg, den_parts, bg, W2, dinv)
    p2 = _sc_segsum_rows(hs2, src_c, dst_c, ew_c, zrow)

    hs3 = _tc(_tc4_body, jax.ShapeDtypeStruct((_N, _D), f32),
              p2, dinv, b2, W3)
    p3 = _sc_segsum_rows(hs3, src_c, dst_c, ew_c, zrow)

    out = _tc(_tc5_body, jax.ShapeDtypeStruct((_N, _D), f32),
              p3, dinv, b3)
    return out


# speedup vs baseline: 1.2069x; 1.2069x over previous
import functools

import jax
import jax.numpy as jnp
from jax import lax
from jax.experimental import pallas as pl
from jax.experimental.pallas import tpu as pltpu
from jax.experimental.pallas import tpu_sc as plsc

_N = 10000
_D = 128
_E_RAW = 320000
_E_TOT = _E_RAW + _N
_NW = 32
_BATCH = 128
_NB = (_E_TOT + _NW * _BATCH - 1) // (_NW * _BATCH)
_E_PAD = _NW * _NB * _BATCH
_N_PAD = 10240
_RPT = _N_PAD // 16
_N_ROW_PAD = 10112
_RPT2 = _N_ROW_PAD // 16

_mesh = plsc.VectorSubcoreMesh(core_axis_name="c", subcore_axis_name="s")


@functools.partial(
    pl.kernel,
    mesh=_mesh,
    compiler_params=pltpu.CompilerParams(needs_layout_passes=False),
    out_type=jax.ShapeDtypeStruct((2, _N_ROW_PAD, _D), jnp.float32),
    scratch_types=[
        pltpu.VMEM((_NB, _BATCH), jnp.int32),
        pltpu.VMEM((_NB, _BATCH), jnp.int32),
        pltpu.VMEM((_NB * _BATCH,), jnp.float32),
        pltpu.VMEM((_BATCH, _D), jnp.float32),
        pltpu.VMEM_SHARED((_N_ROW_PAD, _D), jnp.float32),
        pltpu.SemaphoreType.DMA,
    ],
)
def _sc_segsum_rows(h_hbm, src_hbm, dst_hbm, w_hbm, zero_hbm, part_hbm,
                    src_v, dst_v, w_v, rows_v, acc_sh, sem):
    c = lax.axis_index("c")
    s = lax.axis_index("s")
    wid = c * 16 + s
    pltpu.sync_copy(src_hbm.at[wid], src_v)
    pltpu.sync_copy(dst_hbm.at[wid], dst_v)
    pltpu.sync_copy(w_hbm.at[wid], w_v)
    pltpu.sync_copy(zero_hbm.at[pl.ds(s * _RPT2, _RPT2)],
                    acc_sh.at[pl.ds(s * _RPT2, _RPT2)])
    plsc.subcore_barrier()

    def batch_body(b, carry):
        pltpu.async_copy(h_hbm.at[src_v.at[b]], rows_v, sem).wait()

        def edge_body(j4, carry2):
            j = j4 * 4
            for u in range(4):
                wsp = plsc.load_gather(
                    w_v, [jnp.full((16,), b * _BATCH + j + u, jnp.int32)])
                for t in range(_D // 16):
                    rows_v[j + u, pl.ds(t * 16, 16)] = (
                        rows_v[j + u, pl.ds(t * 16, 16)] * wsp)
            return carry2

        lax.fori_loop(0, _BATCH // 4, edge_body, 0)
        pltpu.sync_copy(rows_v, acc_sh.at[dst_v.at[b]], add=True)
        return carry

    lax.fori_loop(0, _NB, batch_body, 0)
    plsc.subcore_barrier()
    pltpu.sync_copy(acc_sh.at[pl.ds(s * _RPT2, _RPT2)],
                    part_hbm.at[c, pl.ds(s * _RPT2, _RPT2)])


@functools.partial(
    pl.kernel,
    mesh=_mesh,
    compiler_params=pltpu.CompilerParams(needs_layout_passes=False),
    out_type=jax.ShapeDtypeStruct((2, _N_PAD), jnp.float32),
    scratch_types=[
        pltpu.VMEM((_NB, _BATCH), jnp.int32),
        pltpu.VMEM((_NB, _BATCH), jnp.float32),
        pltpu.VMEM_SHARED((_N_PAD,), jnp.float32),
    ],
)
def _sc_degree(dst_hbm, w_hbm, zero_hbm, part_hbm, dst_v, w_v, acc_sh):
    c = lax.axis_index("c")
    s = lax.axis_index("s")
    wid = c * 16 + s
    pltpu.sync_copy(dst_hbm.at[wid], dst_v)
    pltpu.sync_copy(w_hbm.at[wid], w_v)
    pltpu.sync_copy(zero_hbm.at[pl.ds(s * _RPT, _RPT)],
                    acc_sh.at[pl.ds(s * _RPT, _RPT)])
    plsc.subcore_barrier()

    def batch_body(b, carry):
        pltpu.sync_copy(w_v.at[b], acc_sh.at[dst_v.at[b]], add=True)
        return carry

    lax.fori_loop(0, _NB, batch_body, 0)
    plsc.subcore_barrier()
    pltpu.sync_copy(acc_sh.at[pl.ds(s * _RPT, _RPT)],
                    part_hbm.at[c, pl.ds(s * _RPT, _RPT)])


@functools.partial(
    pl.kernel,
    mesh=_mesh,
    compiler_params=pltpu.CompilerParams(needs_layout_passes=False),
    out_type=(
        jax.ShapeDtypeStruct((_NW, _NB, _BATCH), jnp.float32),
        jax.ShapeDtypeStruct((2, _N_PAD), jnp.float32),
    ),
    scratch_types=[
        pltpu.VMEM((_N_PAD,), jnp.float32),
        pltpu.VMEM((_N_PAD,), jnp.float32),
        pltpu.VMEM((16,), jnp.float32),
        pltpu.VMEM((_NB, _BATCH), jnp.int32),
        pltpu.VMEM((_NB, _BATCH), jnp.int32),
        pltpu.VMEM((_NB, _BATCH), jnp.float32),
        pltpu.VMEM((_NB, _BATCH), jnp.float32),
        pltpu.VMEM_SHARED((_N_PAD,), jnp.float32),
    ],
)
def _sc_gat_edges(s1_hbm, s2_hbm, shift_hbm, src_hbm, dst_hbm, mask_hbm,
                  zero_hbm, ex_hbm, dpart_hbm,
                  s1_v, s2_v, shift_v, src_v, dst_v, mask_v, ex_v, acc_sh):
    c = lax.axis_index("c")
    s = lax.axis_index("s")
    wid = c * 16 + s
    pltpu.sync_copy(s1_hbm, s1_v)
    pltpu.sync_copy(s2_hbm, s2_v)
    pltpu.sync_copy(shift_hbm, shift_v)
    pltpu.sync_copy(src_hbm.at[wid], src_v)
    pltpu.sync_copy(dst_hbm.at[wid], dst_v)
    pltpu.sync_copy(mask_hbm.at[wid], mask_v)
    pltpu.sync_copy(zero_hbm.at[pl.ds(s * _RPT, _RPT)],
                    acc_sh.at[pl.ds(s * _RPT, _RPT)])
    plsc.subcore_barrier()
    shift = shift_v[...]

    def batch_body(b, carry):
        def grp_body(k, carry2):
            src16 = src_v[b, pl.ds(k * 16, 16)]
            dst16 = dst_v[b, pl.ds(k * 16, 16)]
            m16 = mask_v[b, pl.ds(k * 16, 16)]
            a = plsc.load_gather(s1_v, [src16]) + plsc.load_gather(s2_v, [dst16])
            a = jnp.where(a > 0.0, a, 0.2 * a) - shift
            ex_v[b, pl.ds(k * 16, 16)] = jnp.exp(a) * m16
            return carry2

        lax.fori_loop(0, _BATCH // 16, grp_body, 0)
        pltpu.sync_copy(ex_v.at[b], acc_sh.at[dst_v.at[b]], add=True)
        return carry

    lax.fori_loop(0, _NB, batch_body, 0)
    pltpu.sync_copy(ex_v, ex_hbm.at[wid])
    plsc.subcore_barrier()
    pltpu.sync_copy(acc_sh.at[pl.ds(s * _RPT, _RPT)],
                    dpart_hbm.at[c, pl.ds(s * _RPT, _RPT)])


def _tc1_body(x_ref, w1_ref, degp_ref, hs1_ref, dinv_ref):
    x = x_ref[...]
    x = jnp.where(jnp.isnan(x) | jnp.isinf(x), jnp.zeros_like(x), x)
    deg = (degp_ref[0, :] + degp_ref[1, :])[:_N]
    dinv = jnp.where(deg > 0.0, lax.rsqrt(deg), 0.0)
    dinv_ref[...] = dinv[:, None]
    hs1_ref[...] = dinv[:, None] * jnp.dot(
        x, w1_ref[...], preferred_element_type=jnp.float32)


def _tc2_body(part_ref, dinv_ref, b1_ref, gamma_ref, beta_ref, wg_ref,
              asrc_ref, adst_ref, hg_ref, s1_ref, s2_ref, shift_ref):
    y = dinv_ref[...] * (part_ref[0] + part_ref[1])[:_N] + b1_ref[...]
    mean = jnp.mean(y, axis=0, keepdims=True)
    var = jnp.mean((y - mean) * (y - mean), axis=0, keepdims=True)
    y = (y - mean) * lax.rsqrt(var + 1e-5) * gamma_ref[...] + beta_ref[...]
    y = jnp.maximum(y, 0.0)
    hg = jnp.dot(y, wg_ref[...], preferred_element_type=jnp.float32)
    hg_ref[...] = hg
    s1 = jnp.dot(hg, asrc_ref[...], preferred_element_type=jnp.float32)
    s2 = jnp.dot(hg, adst_ref[...], preferred_element_type=jnp.float32)
    s1_ref[...] = s1
    s2_ref[...] = s2
    shift_ref[...] = jnp.full((1, 1), 0.0, jnp.float32) + jnp.max(s1) + jnp.max(s2)


def _tc3_body(gpart_ref, dpart_ref, bg_ref, w2_ref, dinv_ref, hs2_ref):
    denom = (dpart_ref[0, :] + dpart_ref[1, :])[:_N, None]
    y = (gpart_ref[0] + gpart_ref[1])[:_N] / denom + bg_ref[...]
    y = jnp.maximum(y, 0.0)
    hs2_ref[...] = dinv_ref[...] * jnp.dot(
        y, w2_ref[...], preferred_element_type=jnp.float32)


def _tc4_body(part_ref, dinv_ref, b2_ref, w3_ref, hs3_ref):
    y = dinv_ref[...] * (part_ref[0] + part_ref[1])[:_N] + b2_ref[...]
    hs3_ref[...] = dinv_ref[...] * jnp.dot(
        y, w3_ref[...], preferred_element_type=jnp.float32)


def _tc5_body(part_ref, dinv_ref, b3_ref, out_ref):
    out_ref[...] = dinv_ref[...] * (part_ref[0] + part_ref[1])[:_N] + b3_ref[...]


def _tc(body, out_shape, *args):
    return pl.pallas_call(body, out_shape=out_shape)(*args)


def kernel(x, edge_index, edge_weight, W1, b1, gamma, beta, Wg, att_src,
           att_dst, bg, W2, b2, W3, b3):
    f32 = jnp.float32
    loop = jnp.arange(_N, dtype=edge_index.dtype)
    src = jnp.concatenate([edge_index[0], loop])
    dst = jnp.concatenate([edge_index[1], loop])
    ew = jnp.concatenate([edge_weight, jnp.ones((_N,), f32)])
    pad = _E_PAD - _E_TOT
    src_c = jnp.pad(src, (0, pad)).reshape(_NW, _NB, _BATCH)
    dst_c = jnp.pad(dst, (0, pad)).reshape(_NW, _NB, _BATCH)
    ew_c = jnp.pad(ew, (0, pad)).reshape(_NW, _NB, _BATCH)
    mask_c = jnp.pad(jnp.ones((_E_TOT,), f32), (0, pad)).reshape(_NW, _NB, _BATCH)
    ew_f = ew_c.reshape(_NW, _NB * _BATCH)
    zrow = jnp.zeros((_N_ROW_PAD, _D), f32)
    zvec = jnp.zeros((_N_PAD,), f32)

    deg_parts = _sc_degree(dst_c, ew_c, zvec)

    hs1, dinv = _tc(
        _tc1_body,
        (jax.ShapeDtypeStruct((_N, _D), f32), jax.ShapeDtypeStruct((_N, 1), f32)),
        x, W1, deg_parts)
    p1 = _sc_segsum_rows(hs1, src_c, dst_c, ew_f, zrow)

    hg, s1, s2, shift = _tc(
        _tc2_body,
        (jax.ShapeDtypeStruct((_N, _D), f32),
         jax.ShapeDtypeStruct((_N,), f32),
         jax.ShapeDtypeStruct((_N,), f32),
         jax.ShapeDtypeStruct((1, 1), f32)),
        p1, dinv, b1, gamma, beta, Wg, att_src, att_dst)
    shift16 = jnp.broadcast_to(shift.reshape(()), (16,))

    s1p = jnp.pad(s1, (0, _N_PAD - _N))
    s2p = jnp.pad(s2, (0, _N_PAD - _N))
    ex_c, den_parts = _sc_gat_edges(s1p, s2p, shift16, src_c, dst_c, mask_c, zvec)
    pg = _sc_segsum_rows(hg, src_c, dst_c, ex_c.reshape(_NW, _NB * _BATCH), zrow)

    hs2 = _tc(_tc3_body, jax.ShapeDtypeStruct((_N, _D), f32),
              pg, den_parts, bg, W2, dinv)
    p2 = _sc_segsum_rows(hs2, src_c, dst_c, ew_f, zrow)

    hs3 = _tc(_tc4_body, jax.ShapeDtypeStruct((_N, _D), f32),
              p2, dinv, b2, W3)
    p3 = _sc_segsum_rows(hs3, src_c, dst_c, ew_f, zrow)

    out = _tc(_tc5_body, jax.ShapeDtypeStruct((_N, _D), f32),
              p3, dinv, b3)
    return out
